# baseline (speedup 1.0000x reference)
import functools
import jax
import jax.numpy as jnp
from jax.experimental import pallas as pl
from jax.experimental.pallas import tpu as pltpu

HW = 256
N_CAND = 64
K_TOP = 9
CP = 64
NEG = -1e30


def _split3_lanes(x):
    hi = x.astype(jnp.bfloat16).astype(jnp.float32)
    r = x - hi
    mid = r.astype(jnp.bfloat16).astype(jnp.float32)
    lo = r - mid
    return jnp.concatenate([hi, mid, lo], axis=1)


def _attn_one(tT, idxr, wT, b, feat_ref, *, scale):
    tTsplit = _split3_lanes(tT)
    row = jax.lax.broadcasted_iota(jnp.int32, (3 * HW, N_CAND), 0)
    ohrepT = ((row % HW) == idxr).astype(jnp.float32)
    candT = jax.lax.dot_general(tTsplit, ohrepT, (((1,), (0,)), ((), ())),
                                preferred_element_type=jnp.float32)
    cand = jax.lax.transpose(candT, (1, 0))

    sT = jax.lax.dot_general(cand, tT, (((1,), (0,)), ((), ())),
                             preferred_element_type=jnp.float32) * scale

    sub = jax.lax.broadcasted_iota(jnp.int32, (N_CAND, HW), 0)
    amins = []
    es = []
    m0 = None
    for k in range(K_TOP):
        m = jnp.max(sT, axis=0, keepdims=True)
        amin = jnp.min(jnp.where(sT == m, sub, N_CAND), axis=0, keepdims=True)
        sel = sub == amin
        if k == 0:
            m0 = m
        es.append(jnp.exp(m - m0))
        amins.append(amin)
        sT = jnp.where(sel, NEG, sT)

    denom = es[0]
    for k in range(1, K_TOP):
        denom = denom + es[k]

    csplitT = _split3_lanes(candT)
    sub3 = jax.lax.broadcasted_iota(jnp.int32, (3 * N_CAND, HW), 0) % N_CAND
    for k in range(K_TOP):
        wk = es[k] / denom
        selrep = (sub3 == amins[k]).astype(jnp.float32)
        nkT = jax.lax.dot_general(csplitT, selrep, (((1,), (0,)), ((), ())),
                                  preferred_element_type=jnp.float32)
        feat_ref[k * CP:(k + 1) * CP, :] = wk * nkT

    accT = jax.lax.dot_general(
        wT, feat_ref[...], (((1,), (0,)), ((), ())),
        preferred_element_type=jnp.float32)
    return jnp.maximum(accT + b, 0.0)


def _layers_body(tokens_ref, idx1_ref, wT1_ref, b1_ref, idx2_ref, wT2_ref,
                 b2_ref, out_ref, feat_ref, *, scale1, scale2, cb):
    idx1r = idx1_ref[...]
    idx2r = idx2_ref[...]
    wT1 = wT1_ref[...]
    wT2 = wT2_ref[...]
    b1 = b1_ref[...]
    b2 = b2_ref[...]
    zpad = jnp.zeros((CP - 12, HW), jnp.float32)
    for i in range(cb):
        tT = jnp.concatenate([tokens_ref[i], zpad], axis=0)
        o1 = _attn_one(tT, idx1r, wT1, b1, feat_ref.at[i % 2], scale=scale1)
        out_ref[i] = _attn_one(o1, idx2r, wT2, b2, feat_ref.at[2 + i % 2],
                               scale=scale2)


def _prep_w(W, Cout):
    C = W.shape[0] // K_TOP
    Wfull = jnp.zeros((K_TOP, CP, Cout), W.dtype)
    Wfull = Wfull.at[:, :C, :].set(W.reshape(K_TOP, C, Cout))
    return Wfull.reshape(K_TOP * CP, Cout).T


def _attn_layers(tokensT, idx1, W1, b1, idx2, W2, b2, cb=8):
    B = tokensT.shape[0]
    WT1 = _prep_w(W1, 64)
    WT2 = _prep_w(W2, 128)
    body = functools.partial(_layers_body, scale1=1.0 / (12.0 ** 0.5),
                             scale2=1.0 / (64.0 ** 0.5), cb=cb)
    rep = lambda i: (0, 0)
    return pl.pallas_call(
        body,
        grid=(B // cb,),
        in_specs=[
            pl.BlockSpec((cb, 12, HW), lambda i: (i, 0, 0)),
            pl.BlockSpec((1, N_CAND), rep),
            pl.BlockSpec((64, K_TOP * CP), rep),
            pl.BlockSpec((64, 1), rep),
            pl.BlockSpec((1, N_CAND), rep),
            pl.BlockSpec((128, K_TOP * CP), rep),
            pl.BlockSpec((128, 1), rep),
        ],
        out_specs=pl.BlockSpec((cb, 128, HW), lambda i: (i, 0, 0)),
        out_shape=jax.ShapeDtypeStruct((B, 128, HW), jnp.float32),
        scratch_shapes=[pltpu.VMEM((4, K_TOP * CP, HW), jnp.float32)],
    )(tokensT, idx1.astype(jnp.int32).reshape(1, N_CAND), WT1,
      b1.reshape(64, 1), idx2.astype(jnp.int32).reshape(1, N_CAND), WT2,
      b2.reshape(128, 1))


def _fc_body(x_ref, w1_ref, b1_ref, w2_ref, b2_ref, out_ref, acc_ref, *, nk):
    k = pl.program_id(0)

    @pl.when(k == 0)
    def _():
        acc_ref[...] = jnp.zeros_like(acc_ref)

    acc_ref[...] += jax.lax.dot_general(
        x_ref[...], w1_ref[...], (((1,), (0,)), ((), ())),
        preferred_element_type=jnp.float32)

    @pl.when(k == nk - 1)
    def _():
        h = jnp.maximum(acc_ref[...] + b1_ref[...], 0.0)
        out_ref[...] = jax.lax.dot_general(
            h, w2_ref[...], (((1,), (0,)), ((), ())),
            preferred_element_type=jnp.float32) + b2_ref[...]


def _fc_head(h, Wf1, bf1, Wf2, bf2, bk=4096):
    B, Kdim = h.shape
    nk = Kdim // bk
    nout = Wf2.shape[1]
    nhid = Wf1.shape[1]
    body = functools.partial(_fc_body, nk=nk)
    return pl.pallas_call(
        body,
        grid=(nk,),
        in_specs=[
            pl.BlockSpec((B, bk), lambda k: (0, k)),
            pl.BlockSpec((bk, nhid), lambda k: (k, 0)),
            pl.BlockSpec((1, nhid), lambda k: (0, 0)),
            pl.BlockSpec((nhid, nout), lambda k: (0, 0)),
            pl.BlockSpec((1, nout), lambda k: (0, 0)),
        ],
        out_specs=pl.BlockSpec((B, nout), lambda k: (0, 0)),
        out_shape=jax.ShapeDtypeStruct((B, nout), jnp.float32),
        scratch_shapes=[pltpu.VMEM((B, nhid), jnp.float32)],
    )(h, Wf1, bf1.reshape(1, nhid), Wf2, bf2.reshape(1, nout))


def kernel(x, idx1, idx2, W1, b1, W2, b2, Wf1, bf1, Wf2, bf2):
    B = x.shape[0]
    t1 = x.reshape(B, 3, 16, 2, 16, 2).transpose(0, 1, 3, 5, 2, 4)
    t1 = t1.reshape(B, 12, HW)

    o2 = _attn_layers(t1, idx1, W1, b1, idx2, W2, b2)

    hflat = o2.reshape(B, 32, 2, 2, 16, 16).transpose(0, 1, 4, 2, 5, 3)
    hflat = hflat.reshape(B, 32 * 32 * 32)

    return _fc_head(hflat, Wf1, bf1, Wf2, bf2)


# speedup vs baseline: 1.0354x; 1.0354x over previous
import functools
import jax
import jax.numpy as jnp
from jax.experimental import pallas as pl
from jax.experimental.pallas import tpu as pltpu

HW = 256
N_CAND = 64
K_TOP = 9
CP = 64
NEG = -1e30


def _split3_lanes(x):
    hi = x.astype(jnp.bfloat16).astype(jnp.float32)
    r = x - hi
    mid = r.astype(jnp.bfloat16).astype(jnp.float32)
    lo = r - mid
    return jnp.concatenate([hi, mid, lo], axis=1)


def _attn_one(tT, idxr, wT, b, *, scale, cc):
    tTsplit = _split3_lanes(tT)
    row = jax.lax.broadcasted_iota(jnp.int32, (3 * HW, N_CAND), 0)
    ohrepT = ((row % HW) == idxr).astype(jnp.float32)
    candT = jax.lax.dot_general(tTsplit, ohrepT, (((1,), (0,)), ((), ())),
                                preferred_element_type=jnp.float32)
    cand = jax.lax.transpose(candT, (1, 0))

    sT = jax.lax.dot_general(cand, tT, (((1,), (0,)), ((), ())),
                             preferred_element_type=jnp.float32) * scale

    sub = jax.lax.broadcasted_iota(jnp.int32, (N_CAND, HW), 0)
    sels = []
    es = []
    m0 = None
    for k in range(K_TOP):
        m = jnp.max(sT, axis=0, keepdims=True)
        amin = jnp.min(jnp.where(sT == m, sub, N_CAND), axis=0, keepdims=True)
        sel = sub == amin
        if k == 0:
            m0 = m
        es.append(jnp.exp(m - m0))
        sels.append(sel.astype(jnp.float32))
        sT = jnp.where(sel, NEG, sT)

    t0 = es[0] + es[8]
    t0 = t0 + es[4]
    t0 = t0 + (es[2] + es[6])
    t1 = (es[1] + es[5]) + (es[3] + es[7])
    denom = t0 + t1

    csplitT = _split3_lanes(candT[:cc, :])
    slabs = []
    for k in range(K_TOP):
        wk = es[k] / denom
        selrep = jnp.concatenate([sels[k]] * 3, axis=0)
        nkT = jax.lax.dot_general(csplitT, selrep, (((1,), (0,)), ((), ())),
                                  preferred_element_type=jnp.float32)
        slabs.append(wk * nkT)
    featT = jnp.concatenate(slabs, axis=0)

    accT = jax.lax.dot_general(
        wT, featT, (((1,), (0,)), ((), ())),
        preferred_element_type=jnp.float32)
    return jnp.maximum(accT + b, 0.0)


def _layers_body(tokens_ref, idx1_ref, wT1_ref, b1_ref, idx2_ref, wT2_ref,
                 b2_ref, out_ref, *, scale1, scale2, cb):
    idx1r = idx1_ref[...]
    idx2r = idx2_ref[...]
    wT1 = wT1_ref[...]
    wT2 = wT2_ref[...]
    b1 = b1_ref[...]
    b2 = b2_ref[...]
    zpad = jnp.zeros((CP - 12, HW), jnp.float32)
    for i in range(cb):
        tT = jnp.concatenate([tokens_ref[i], zpad], axis=0)
        o1 = _attn_one(tT, idx1r, wT1, b1, scale=scale1, cc=12)
        out_ref[i] = _attn_one(o1, idx2r, wT2, b2, scale=scale2, cc=64)


def _attn_layers(tokensT, idx1, W1, b1, idx2, W2, b2, cb=8):
    B = tokensT.shape[0]
    WT1 = W1.T
    WT2 = W2.T
    body = functools.partial(_layers_body, scale1=1.0 / (12.0 ** 0.5),
                             scale2=1.0 / (64.0 ** 0.5), cb=cb)
    rep = lambda i: (0, 0)
    return pl.pallas_call(
        body,
        grid=(B // cb,),
        in_specs=[
            pl.BlockSpec((cb, 12, HW), lambda i: (i, 0, 0)),
            pl.BlockSpec((1, N_CAND), rep),
            pl.BlockSpec((64, K_TOP * 12), rep),
            pl.BlockSpec((64, 1), rep),
            pl.BlockSpec((1, N_CAND), rep),
            pl.BlockSpec((128, K_TOP * CP), rep),
            pl.BlockSpec((128, 1), rep),
        ],
        out_specs=pl.BlockSpec((cb, 128, HW), lambda i: (i, 0, 0)),
        out_shape=jax.ShapeDtypeStruct((B, 128, HW), jnp.float32),
    )(tokensT, idx1.astype(jnp.int32).reshape(1, N_CAND), WT1,
      b1.reshape(64, 1), idx2.astype(jnp.int32).reshape(1, N_CAND), WT2,
      b2.reshape(128, 1))


def _fc_body(x_ref, w1_ref, b1_ref, w2_ref, b2_ref, out_ref, acc_ref, *, nk):
    k = pl.program_id(0)

    @pl.when(k == 0)
    def _():
        acc_ref[...] = jnp.zeros_like(acc_ref)

    acc_ref[...] += jax.lax.dot_general(
        x_ref[...], w1_ref[...], (((1,), (0,)), ((), ())),
        preferred_element_type=jnp.float32)

    @pl.when(k == nk - 1)
    def _():
        h = jnp.maximum(acc_ref[...] + b1_ref[...], 0.0)
        out_ref[...] = jax.lax.dot_general(
            h, w2_ref[...], (((1,), (0,)), ((), ())),
            preferred_element_type=jnp.float32) + b2_ref[...]


def _fc_head(h, Wf1, bf1, Wf2, bf2, bk=4096):
    B, Kdim = h.shape
    nk = Kdim // bk
    nout = Wf2.shape[1]
    nhid = Wf1.shape[1]
    body = functools.partial(_fc_body, nk=nk)
    return pl.pallas_call(
        body,
        grid=(nk,),
        in_specs=[
            pl.BlockSpec((B, bk), lambda k: (0, k)),
            pl.BlockSpec((bk, nhid), lambda k: (k, 0)),
            pl.BlockSpec((1, nhid), lambda k: (0, 0)),
            pl.BlockSpec((nhid, nout), lambda k: (0, 0)),
            pl.BlockSpec((1, nout), lambda k: (0, 0)),
        ],
        out_specs=pl.BlockSpec((B, nout), lambda k: (0, 0)),
        out_shape=jax.ShapeDtypeStruct((B, nout), jnp.float32),
        scratch_shapes=[pltpu.VMEM((B, nhid), jnp.float32)],
    )(h, Wf1, bf1.reshape(1, nhid), Wf2, bf2.reshape(1, nout))


def kernel(x, idx1, idx2, W1, b1, W2, b2, Wf1, bf1, Wf2, bf2):
    B = x.shape[0]
    t1 = x.reshape(B, 3, 16, 2, 16, 2).transpose(0, 1, 3, 5, 2, 4)
    t1 = t1.reshape(B, 12, HW)

    o2 = _attn_layers(t1, idx1, W1, b1, idx2, W2, b2)

    hflat = o2.reshape(B, 32, 2, 2, 16, 16).transpose(0, 1, 4, 2, 5, 3)
    hflat = hflat.reshape(B, 32 * 32 * 32)

    return _fc_head(hflat, Wf1, bf1, Wf2, bf2)
